# baseline (speedup 1.0000x reference)
import functools

import jax
import jax.numpy as jnp
from jax import lax
from jax.experimental import pallas as pl
from jax.experimental.pallas import tpu as pltpu
from jax.experimental.pallas import tpu_sc as plsc

VOCAB = 100000
D = 128
NUM_CTX = 5
B = 4096
NW = 32
B_PER_W = B // NW
L = 16
NCHUNK = D // L

_CHUNK_READY_B = [(c * B_PER_W) // NUM_CTX + 1 for c in range(1, NUM_CTX)]


def _sc_body(tgt_idx_hbm, ctx_idx_hbm, wt_hbm, wc_hbm, out_hbm,
             idx_t3, idx_c3, idx_t, idx_c, te, ce, matv, out_v,
             sem_t, *sem_c):
    cid = lax.axis_index("c")
    sid = lax.axis_index("s")
    wid = sid * 2 + cid
    base = wid * B_PER_W

    pltpu.sync_copy(tgt_idx_hbm.at[pl.ds(base, B_PER_W)], idx_t3)
    pltpu.sync_copy(ctx_idx_hbm.at[pl.ds(base, B_PER_W)], idx_c3)

    lanes = lax.iota(jnp.int32, L)
    zeros = jnp.zeros((L,), jnp.int32)
    for t in range(B_PER_W // L):
        idx_t[pl.ds(t * L, L)] = plsc.load_gather(
            idx_t3, [t * L + lanes, zeros])
    for t in range((B_PER_W * NUM_CTX) // L):
        f = t * L + lanes
        bq = (f * 52429) >> 18
        jr = f - bq * NUM_CTX
        idx_c[pl.ds(t * L, L)] = plsc.load_gather(idx_c3, [bq, jr, zeros])

    cp_t = pltpu.make_async_copy(wt_hbm.at[idx_t], te, sem_t)
    cp_c = [
        pltpu.make_async_copy(
            wc_hbm.at[idx_c.at[pl.ds(c * B_PER_W, B_PER_W)]],
            ce.at[pl.ds(c * B_PER_W, B_PER_W)],
            sem_c[c],
        )
        for c in range(NUM_CTX)
    ]
    cp_t.start()
    for cp in cp_c:
        cp.start()
    cp_t.wait()
    cp_c[0].wait()

    def body_a(b, carry):
        for c in range(1, NUM_CTX):
            @pl.when(b == _CHUNK_READY_B[c - 1])
            def _wait():
                cp_c[c].wait()
        tch = [te[b, pl.ds(k * L, L)] for k in range(NCHUNK)]
        for j in range(NUM_CTX):
            row = b * NUM_CTX + j
            prod = [ce[row, pl.ds(k * L, L)] * tch[k] for k in range(NCHUNK)]
            while len(prod) > 1:
                prod = [prod[2 * m] + prod[2 * m + 1]
                        for m in range(len(prod) // 2)]
            matv[j * B_PER_W + b, pl.ds(0, L)] = prod[0]
        return carry

    lax.fori_loop(0, B_PER_W, body_a, 0)

    lanes = lax.iota(jnp.int32, L)
    cols = [jnp.full((L,), d, jnp.int32) for d in range(L)]

    def body_b(g, carry):
        rows = g * L + lanes
        s = plsc.load_gather(matv, [rows, cols[0]])
        for d in range(1, L):
            s = s + plsc.load_gather(matv, [rows, cols[d]])
        bvec = ((g & 7) << 4) + lanes
        jvec = jnp.zeros((L,), jnp.int32) + (g >> 3)
        plsc.store_scatter(out_v, [bvec, jvec], s)
        return carry

    lax.fori_loop(0, (B_PER_W * NUM_CTX) // L, body_b, 0)

    pltpu.sync_copy(out_v, out_hbm.at[pl.ds(base, B_PER_W)])


@jax.jit
def _act2vec_sc(tgt_idx, ctx_idx, W_target, W_context):
    mesh = plsc.VectorSubcoreMesh(core_axis_name="c", subcore_axis_name="s")
    kern = functools.partial(
        pl.kernel,
        mesh=mesh,
        out_type=jax.ShapeDtypeStruct((B, NUM_CTX), jnp.float32),
        scratch_types=[
            pltpu.VMEM((B_PER_W, 1), jnp.int32),
            pltpu.VMEM((B_PER_W, NUM_CTX, 1), jnp.int32),
            pltpu.VMEM((B_PER_W,), jnp.int32),
            pltpu.VMEM((B_PER_W * NUM_CTX,), jnp.int32),
            pltpu.VMEM((B_PER_W, D), jnp.float32),
            pltpu.VMEM((NUM_CTX * B_PER_W, D), jnp.float32),
            pltpu.VMEM((B_PER_W * NUM_CTX, L), jnp.float32),
            pltpu.VMEM((B_PER_W, NUM_CTX), jnp.float32),
            pltpu.SemaphoreType.DMA,
            pltpu.SemaphoreType.DMA,
            pltpu.SemaphoreType.DMA,
            pltpu.SemaphoreType.DMA,
            pltpu.SemaphoreType.DMA,
            pltpu.SemaphoreType.DMA,
        ],
        compiler_params=pltpu.CompilerParams(
            needs_layout_passes=False, use_tc_tiling_on_sc=False),
    )(_sc_body)
    return kern(tgt_idx, ctx_idx, W_target, W_context)


def kernel(target, context, W_target, W_context):
    return _act2vec_sc(target.astype(jnp.int32), context.astype(jnp.int32),
                       W_target, W_context)


# speedup vs baseline: 1.7844x; 1.7844x over previous
import functools

import jax
import jax.numpy as jnp
from jax import lax
from jax.experimental import pallas as pl
from jax.experimental.pallas import tpu as pltpu
from jax.experimental.pallas import tpu_sc as plsc

VOCAB = 100000
D = 128
NUM_CTX = 5
B = 4096
NW = 32
B_PER_W = B // NW
L = 16
NCHUNK = D // L

_CHUNK_READY_B = [(c * B_PER_W) // NUM_CTX + 1 for c in range(1, NUM_CTX)]


def _sc_body(tgt_idx_hbm, ctx_idx_hbm, wt_hbm, wc_hbm, out_hbm,
             idx_t, idx_c, te, ce, matv, out_v, sem_t, *sem_c):
    cid = lax.axis_index("c")
    sid = lax.axis_index("s")
    wid = sid * 2 + cid
    base = wid * B_PER_W

    pltpu.sync_copy(tgt_idx_hbm.at[pl.ds(base, B_PER_W)], idx_t)
    pltpu.sync_copy(
        ctx_idx_hbm.at[pl.ds(base * NUM_CTX, B_PER_W * NUM_CTX)], idx_c)

    cp_t = pltpu.make_async_copy(wt_hbm.at[idx_t], te, sem_t)
    cp_c = [
        pltpu.make_async_copy(
            wc_hbm.at[idx_c.at[pl.ds(c * B_PER_W, B_PER_W)]],
            ce.at[pl.ds(c * B_PER_W, B_PER_W)],
            sem_c[c],
        )
        for c in range(NUM_CTX)
    ]
    cp_t.start()
    for cp in cp_c:
        cp.start()
    cp_t.wait()
    cp_c[0].wait()

    UA = 2

    def body_a(it, carry):
        for c in range(1, NUM_CTX):
            @pl.when(it == _CHUNK_READY_B[c - 1] // UA)
            def _wait():
                cp_c[c].wait()
        for u in range(UA):
            b = it * UA + u
            tch = [te[b, pl.ds(k * L, L)] for k in range(NCHUNK)]
            for j in range(NUM_CTX):
                row = b * NUM_CTX + j
                prod = [ce[row, pl.ds(k * L, L)] * tch[k]
                        for k in range(NCHUNK)]
                while len(prod) > 1:
                    prod = [prod[2 * m] + prod[2 * m + 1]
                            for m in range(len(prod) // 2)]
                matv[pl.ds((j * B_PER_W + b) * L, L)] = prod[0]
        return carry

    lax.fori_loop(0, B_PER_W // UA, body_a, 0)

    lanes = lax.iota(jnp.int32, L)
    cols = [lanes * L + d for d in range(L)]
    BLK = L * L

    def body_b(g, carry):
        blk = matv.at[pl.ds(g * BLK, BLK)]
        s = plsc.load_gather(blk, [cols[0]])
        for d in range(1, L):
            s = s + plsc.load_gather(blk, [cols[d]])
        bvec = ((g & 7) << 4) + lanes
        jvec = jnp.zeros((L,), jnp.int32) + (g >> 3)
        plsc.store_scatter(out_v, [bvec, jvec], s)
        return carry

    lax.fori_loop(0, (B_PER_W * NUM_CTX) // L, body_b, 0)

    pltpu.sync_copy(out_v, out_hbm.at[pl.ds(base, B_PER_W)])


@jax.jit
def _act2vec_sc(tgt_idx, ctx_idx, W_target, W_context):
    mesh = plsc.VectorSubcoreMesh(core_axis_name="c", subcore_axis_name="s")
    kern = functools.partial(
        pl.kernel,
        mesh=mesh,
        out_type=jax.ShapeDtypeStruct((B, NUM_CTX), jnp.float32),
        scratch_types=[
            pltpu.VMEM((B_PER_W,), jnp.int32),
            pltpu.VMEM((B_PER_W * NUM_CTX,), jnp.int32),
            pltpu.VMEM((B_PER_W, D), jnp.float32),
            pltpu.VMEM((NUM_CTX * B_PER_W, D), jnp.float32),
            pltpu.VMEM((B_PER_W * NUM_CTX * L,), jnp.float32),
            pltpu.VMEM((B_PER_W, NUM_CTX), jnp.float32),
            pltpu.SemaphoreType.DMA,
            pltpu.SemaphoreType.DMA,
            pltpu.SemaphoreType.DMA,
            pltpu.SemaphoreType.DMA,
            pltpu.SemaphoreType.DMA,
            pltpu.SemaphoreType.DMA,
        ],
        compiler_params=pltpu.CompilerParams(
            needs_layout_passes=False, use_tc_tiling_on_sc=False),
    )(_sc_body)
    return kern(tgt_idx, ctx_idx, W_target, W_context)


def kernel(target, context, W_target, W_context):
    tgt_idx = target.reshape(B).astype(jnp.int32)
    ctx_idx = context.reshape(B * NUM_CTX).astype(jnp.int32)
    return _act2vec_sc(tgt_idx, ctx_idx, W_target, W_context)


# speedup vs baseline: 1.8788x; 1.0529x over previous
import functools

import jax
import jax.numpy as jnp
from jax import lax
from jax.experimental import pallas as pl
from jax.experimental.pallas import tpu as pltpu
from jax.experimental.pallas import tpu_sc as plsc

VOCAB = 100000
D = 128
NUM_CTX = 5
B = 4096
NW = 32
B_PER_W = B // NW
L = 16
NCHUNK = D // L

_CHUNK_READY_B = [(c * B_PER_W) // NUM_CTX + 1 for c in range(1, NUM_CTX)]


def _sc_body(tgt_idx_hbm, ctx_idx_hbm, wt_hbm, wc_hbm, out_hbm,
             idx_t, idx_c, te, ce, matv, out_v, sem_t, *sem_c):
    cid = lax.axis_index("c")
    sid = lax.axis_index("s")
    wid = sid * 2 + cid
    base = wid * B_PER_W

    pltpu.sync_copy(tgt_idx_hbm.at[pl.ds(base, B_PER_W)], idx_t)
    pltpu.sync_copy(
        ctx_idx_hbm.at[pl.ds(base * NUM_CTX, B_PER_W * NUM_CTX)], idx_c)

    cp_t = pltpu.make_async_copy(wt_hbm.at[idx_t], te, sem_t)
    cp_c = [
        pltpu.make_async_copy(
            wc_hbm.at[idx_c.at[pl.ds(c * B_PER_W, B_PER_W)]],
            ce.at[pl.ds(c * B_PER_W, B_PER_W)],
            sem_c[c],
        )
        for c in range(NUM_CTX)
    ]
    cp_t.start()
    cp_c[0].start()
    cp_t.wait()
    cp_c[0].wait()
    cp_c[1].start()
    cp_c[2].start()

    UA = 2

    def body_a(it, carry):
        for c in range(1, NUM_CTX):
            @pl.when(it == _CHUNK_READY_B[c - 1] // UA)
            def _wait():
                cp_c[c].wait()
                if c + 2 < NUM_CTX:
                    cp_c[c + 2].start()
        for u in range(UA):
            b = it * UA + u
            tch = [te[b, pl.ds(k * L, L)] for k in range(NCHUNK)]
            for j in range(NUM_CTX):
                row = b * NUM_CTX + j
                prod = [ce[row, pl.ds(k * L, L)] * tch[k]
                        for k in range(NCHUNK)]
                while len(prod) > 1:
                    prod = [prod[2 * m] + prod[2 * m + 1]
                            for m in range(len(prod) // 2)]
                matv[pl.ds((j * B_PER_W + b) * L, L)] = prod[0]
        return carry

    lax.fori_loop(0, B_PER_W // UA, body_a, 0)

    lanes = lax.iota(jnp.int32, L)
    cols = [lanes * L + d for d in range(L)]
    BLK = L * L

    def body_b(g, carry):
        blk = matv.at[pl.ds(g * BLK, BLK)]
        acc = [plsc.load_gather(blk, [cols[d]]) for d in range(L)]
        while len(acc) > 1:
            acc = [acc[2 * m] + acc[2 * m + 1] for m in range(len(acc) // 2)]
        s = acc[0]
        bvec = ((g & 7) << 4) + lanes
        jvec = jnp.zeros((L,), jnp.int32) + (g >> 3)
        plsc.store_scatter(out_v, [bvec, jvec], s)
        return carry

    lax.fori_loop(0, (B_PER_W * NUM_CTX) // L, body_b, 0)

    pltpu.sync_copy(out_v, out_hbm.at[pl.ds(base, B_PER_W)])


@jax.jit
def _act2vec_sc(tgt_idx, ctx_idx, W_target, W_context):
    mesh = plsc.VectorSubcoreMesh(core_axis_name="c", subcore_axis_name="s")
    kern = functools.partial(
        pl.kernel,
        mesh=mesh,
        out_type=jax.ShapeDtypeStruct((B, NUM_CTX), jnp.float32),
        scratch_types=[
            pltpu.VMEM((B_PER_W,), jnp.int32),
            pltpu.VMEM((B_PER_W * NUM_CTX,), jnp.int32),
            pltpu.VMEM((B_PER_W, D), jnp.float32),
            pltpu.VMEM((NUM_CTX * B_PER_W, D), jnp.float32),
            pltpu.VMEM((B_PER_W * NUM_CTX * L,), jnp.float32),
            pltpu.VMEM((B_PER_W, NUM_CTX), jnp.float32),
            pltpu.SemaphoreType.DMA,
            pltpu.SemaphoreType.DMA,
            pltpu.SemaphoreType.DMA,
            pltpu.SemaphoreType.DMA,
            pltpu.SemaphoreType.DMA,
            pltpu.SemaphoreType.DMA,
        ],
        compiler_params=pltpu.CompilerParams(
            needs_layout_passes=False, use_tc_tiling_on_sc=False),
    )(_sc_body)
    return kern(tgt_idx, ctx_idx, W_target, W_context)


def kernel(target, context, W_target, W_context):
    tgt_idx = target.reshape(B).astype(jnp.int32)
    ctx_idx = context.reshape(B * NUM_CTX).astype(jnp.int32)
    return _act2vec_sc(tgt_idx, ctx_idx, W_target, W_context)
